# baseline (speedup 1.0000x reference)
import functools

import jax
import jax.numpy as jnp
from jax import lax
from jax.experimental import pallas as pl
from jax.experimental.pallas import tpu as pltpu
from jax.experimental.pallas import tpu_sc as plsc

N = 10000
NP = 10240
E = 320000
D = 64

NC = 2
NS = 16
NW = NC * NS
CH = 128
EP = 327680
NCH = EP // (NW * CH)
NB = 4
RPS = NP // NS

_MESH = plsc.VectorSubcoreMesh(core_axis_name="c", subcore_axis_name="s")
_SC_PARAMS = pltpu.CompilerParams(use_tc_tiling_on_sc=False)


def _zero_vmem_f32(ref, nrows, width):
    z16 = jnp.zeros((16,), jnp.float32)

    def body(i, c):
        for j in range(width // 16):
            ref[i, pl.ds(j * 16, 16)] = z16
        return c

    lax.fori_loop(0, nrows, body, 0)


@functools.partial(
    pl.kernel,
    out_type=jax.ShapeDtypeStruct((NC, NP, 16), jnp.float32),
    mesh=_MESH,
    compiler_params=_SC_PARAMS,
    scratch_types=[
        pltpu.VMEM((NCH, CH), jnp.int32),
        pltpu.VMEM((NCH, CH), jnp.int32),
        pltpu.VMEM((CH, 16), jnp.float32),
        pltpu.VMEM((CH, 16), jnp.float32),
        pltpu.VMEM((RPS // 5, 16), jnp.float32),
        pltpu.VMEM_SHARED((NP, 16), jnp.float32),
        pltpu.SemaphoreType.DMA,
        pltpu.SemaphoreType.DMA,
    ],
)
def _hist(src_hbm, dst_hbm, out_hbm, src_v, dst_v, ones_s, ones_d, zbuf, acc,
          sem_s, sem_d):
    cid = lax.axis_index("c")
    sid = lax.axis_index("s")
    wid = cid * NS + sid

    lane = lax.iota(jnp.int32, 16)
    row_s = jnp.where(lane < 8, 1.0, 0.0)
    row_d = jnp.where(lane < 8, 0.0, 1.0)

    def fill_ones(i, c):
        ones_s[i, pl.ds(0, 16)] = row_s
        ones_d[i, pl.ds(0, 16)] = row_d
        return c

    lax.fori_loop(0, CH, fill_ones, 0)
    _zero_vmem_f32(zbuf, RPS // 5, 16)

    for k in range(5):
        pltpu.sync_copy(
            zbuf, acc.at[pl.ds(sid * RPS + k * (RPS // 5), RPS // 5)])

    pltpu.sync_copy(src_hbm.at[wid], src_v)
    pltpu.sync_copy(dst_hbm.at[wid], dst_v)

    plsc.subcore_barrier()

    WIN = 4

    def chunk(j, c):
        pltpu.async_copy(ones_s, acc.at[src_v.at[j]], sem_s, add=True)
        pltpu.async_copy(ones_d, acc.at[dst_v.at[j]], sem_d, add=True)

        @pl.when(j >= WIN)
        def _():
            pltpu.make_async_copy(ones_s, acc.at[src_v.at[0]], sem_s).wait()
            pltpu.make_async_copy(ones_d, acc.at[dst_v.at[0]], sem_d).wait()

        return c

    lax.fori_loop(0, NCH, chunk, 0)
    for _ in range(WIN):
        pltpu.make_async_copy(ones_s, acc.at[src_v.at[0]], sem_s).wait()
        pltpu.make_async_copy(ones_d, acc.at[dst_v.at[0]], sem_d).wait()

    plsc.subcore_barrier()

    r0 = sid * RPS
    pltpu.sync_copy(acc.at[pl.ds(r0, RPS)], out_hbm.at[cid, pl.ds(r0, RPS)])


@functools.partial(
    pl.kernel,
    out_type=jax.ShapeDtypeStruct((NC, NP, D), jnp.float32),
    mesh=_MESH,
    compiler_params=_SC_PARAMS,
    scratch_types=[
        pltpu.VMEM((NCH, CH), jnp.int32),
        pltpu.VMEM((NCH, CH), jnp.int32),
        [[pltpu.VMEM((CH, D), jnp.float32)] * NB] * 2,
        pltpu.VMEM_SHARED((NP, D), jnp.float32),
        [[pltpu.SemaphoreType.DMA] * NB] * 2,
        [[pltpu.SemaphoreType.DMA] * NB] * 2,
    ],
)
def _agg(y_hbm, src_hbm, dst_hbm, out_hbm, src_v, dst_v, rows, acc,
         gsems, ssems):
    cid = lax.axis_index("c")
    sid = lax.axis_index("s")
    wid = cid * NS + sid

    _zero_vmem_f32(rows[0][0], RPS // 5, D)
    for k in range(5):
        pltpu.sync_copy(
            rows[0][0], acc.at[pl.ds(sid * RPS + k * (RPS // 5), RPS // 5)])

    pltpu.sync_copy(src_hbm.at[wid], src_v)
    pltpu.sync_copy(dst_hbm.at[wid], dst_v)

    plsc.subcore_barrier()

    def drain(p):
        for b in range(NB):
            pltpu.make_async_copy(rows[p][b], acc.at[dst_v.at[0]],
                                  ssems[p][b]).wait()

    def run_group(g, p):
        base = g * NB
        gds = [pltpu.async_copy(y_hbm.at[src_v.at[base + b]], rows[p][b],
                                gsems[p][b]) for b in range(NB)]
        for b in range(NB):
            gds[b].wait()
            pltpu.async_copy(rows[p][b], acc.at[dst_v.at[base + b]],
                             ssems[p][b], add=True)

    NG = NCH // NB

    def pair(i, c):
        for p in range(2):
            @pl.when(i > 0)
            def _():
                drain(p)

            run_group(2 * i + p, p)
        return c

    lax.fori_loop(0, NG // 2, pair, 0)
    drain(0)
    drain(1)

    plsc.subcore_barrier()

    r0 = sid * RPS
    pltpu.sync_copy(acc.at[pl.ds(r0, RPS)], out_hbm.at[cid, pl.ds(r0, RPS)])


_BLK = 1024
_GRID = NP // _BLK


def _tc0_body(hx_ref, ns_ref, w_ref, y_ref):
    x = hx_ref[...] * ns_ref[...]
    y_ref[...] = jnp.dot(x, w_ref[...], preferred_element_type=jnp.float32)


def _tc0(hx, ns, W1):
    return pl.pallas_call(
        _tc0_body,
        out_shape=jax.ShapeDtypeStruct((NP, D), jnp.float32),
        grid=(_GRID,),
        in_specs=[
            pl.BlockSpec((_BLK, 128), lambda i: (i, 0)),
            pl.BlockSpec((_BLK, 1), lambda i: (i, 0)),
            pl.BlockSpec((128, D), lambda i: (0, 0)),
        ],
        out_specs=pl.BlockSpec((_BLK, D), lambda i: (i, 0)),
    )(hx, ns, W1)


def _layer_body(p_ref, w_ref, b_ref, ns_ref, nd_ref, y_ref):
    p = p_ref[0] + p_ref[1]
    p = jnp.dot(p, w_ref[...], preferred_element_type=jnp.float32)
    h = jnp.maximum(p * nd_ref[...] + b_ref[...], 0.0)
    y_ref[...] = h * ns_ref[...]


def _layer(partials, W, b, ns, nd):
    return pl.pallas_call(
        _layer_body,
        out_shape=jax.ShapeDtypeStruct((NP, D), jnp.float32),
        grid=(_GRID,),
        in_specs=[
            pl.BlockSpec((NC, _BLK, D), lambda i: (0, i, 0)),
            pl.BlockSpec((D, D), lambda i: (0, 0)),
            pl.BlockSpec((1, D), lambda i: (0, 0)),
            pl.BlockSpec((_BLK, 1), lambda i: (i, 0)),
            pl.BlockSpec((_BLK, 1), lambda i: (i, 0)),
        ],
        out_specs=pl.BlockSpec((_BLK, D), lambda i: (i, 0)),
    )(partials, W, b, ns, nd)


def _head_body(h_ref, inv_ref, wp1_ref, bp1_ref, wp2_ref, bp2_ref, out_ref):
    rows = lax.broadcasted_iota(jnp.int32, (NP, 1), 0)
    h = jnp.where(rows < N, h_ref[...] * inv_ref[...], 0.0)
    m = jnp.sum(h, axis=0, keepdims=True) * (1.0 / N)
    t = jnp.maximum(jnp.dot(m, wp1_ref[...], preferred_element_type=jnp.float32)
                    + bp1_ref[...], 0.0)
    out_ref[...] = jnp.dot(t, wp2_ref[...], preferred_element_type=jnp.float32) \
        + bp2_ref[...]


def _head(h, inv_ns, Wp1, bp1, Wp2, bp2):
    return pl.pallas_call(
        _head_body,
        out_shape=jax.ShapeDtypeStruct((1, 10), jnp.float32),
    )(h, inv_ns, Wp1, bp1, Wp2, bp2)


def kernel(hx, edge_index, W1, b1, W2, b2, W3, b3, W4, b4, W5, b5, W6, b6,
           Wp1, bp1, Wp2, bp2):
    pad_src = jnp.full((EP - E,), N, jnp.int32)
    pad_dst = jnp.full((EP - E,), NP - 1, jnp.int32)
    src3d = jnp.concatenate([edge_index[0], pad_src]).reshape(NW, NCH, CH)
    dst3d = jnp.concatenate([edge_index[1], pad_dst]).reshape(NW, NCH, CH)
    hxp = jnp.concatenate(
        [hx, jnp.zeros((NP - N, hx.shape[1]), jnp.float32)], axis=0)

    degp = _hist(src3d, dst3d)
    deg_out = degp[0, :, 0] + degp[1, :, 0]
    deg_in = degp[0, :, 8] + degp[1, :, 8]
    ns = lax.rsqrt(jnp.maximum(deg_out, 1.0)).reshape(NP, 1)
    nd = lax.rsqrt(jnp.maximum(deg_in, 1.0)).reshape(NP, 1)

    y = _tc0(hxp, ns, W1)

    Wstack = jnp.stack([jnp.eye(D, dtype=jnp.float32), W2, W3, W4, W5, W6])
    bstack = jnp.stack([b.reshape(1, D) for b in (b1, b2, b3, b4, b5, b6)])

    def step(y, wb):
        W, b = wb
        partials = _agg(y, src3d, dst3d)
        return _layer(partials, W, b, ns, nd), None

    y, _ = lax.scan(step, y, (Wstack, bstack))

    inv_ns = jnp.sqrt(jnp.maximum(deg_out, 1.0)).reshape(NP, 1)
    return _head(y, inv_ns, Wp1, bp1.reshape(1, D // 2), Wp2, bp2.reshape(1, 10))


# speedup vs baseline: 2.6675x; 2.6675x over previous
import functools

import jax
import jax.numpy as jnp
from jax import lax
from jax.experimental import pallas as pl
from jax.experimental.pallas import tpu as pltpu
from jax.experimental.pallas import tpu_sc as plsc

N = 10000
NP = 10240
E = 320000
D = 64

NC = 2
NS = 16
NW = NC * NS
CH = 80
NCH = E // (NW * CH)
NB = 5
RPS = NP // NS

_MESH = plsc.VectorSubcoreMesh(core_axis_name="c", subcore_axis_name="s")
_SC_PARAMS = pltpu.CompilerParams(use_tc_tiling_on_sc=False)


def _zero_vmem_f32(ref, nrows, width):
    z16 = jnp.zeros((16,), jnp.float32)

    def body(i, c):
        for j in range(width // 16):
            ref[i, pl.ds(j * 16, 16)] = z16
        return c

    lax.fori_loop(0, nrows, body, 0)


@functools.partial(
    pl.kernel,
    out_type=jax.ShapeDtypeStruct((NC, NP, 16), jnp.float32),
    mesh=_MESH,
    compiler_params=_SC_PARAMS,
    scratch_types=[
        pltpu.VMEM((NCH, CH), jnp.int32),
        pltpu.VMEM((NCH, CH), jnp.int32),
        pltpu.VMEM((CH, 16), jnp.float32),
        pltpu.VMEM((CH, 16), jnp.float32),
        pltpu.VMEM((RPS // 5, 16), jnp.float32),
        pltpu.VMEM_SHARED((NP, 16), jnp.float32),
        pltpu.SemaphoreType.DMA,
        pltpu.SemaphoreType.DMA,
    ],
)
def _hist(src_hbm, dst_hbm, out_hbm, src_v, dst_v, ones_s, ones_d, zbuf, acc,
          sem_s, sem_d):
    cid = lax.axis_index("c")
    sid = lax.axis_index("s")
    wid = cid * NS + sid

    lane = lax.iota(jnp.int32, 16)
    row_s = jnp.where(lane < 8, 1.0, 0.0)
    row_d = jnp.where(lane < 8, 0.0, 1.0)

    def fill_ones(i, c):
        ones_s[i, pl.ds(0, 16)] = row_s
        ones_d[i, pl.ds(0, 16)] = row_d
        return c

    lax.fori_loop(0, CH, fill_ones, 0)
    _zero_vmem_f32(zbuf, RPS // 5, 16)

    for k in range(5):
        pltpu.sync_copy(
            zbuf, acc.at[pl.ds(sid * RPS + k * (RPS // 5), RPS // 5)])

    pltpu.sync_copy(src_hbm.at[wid], src_v)
    pltpu.sync_copy(dst_hbm.at[wid], dst_v)

    plsc.subcore_barrier()

    WIN = 4

    def chunk(j, c):
        pltpu.async_copy(ones_s, acc.at[src_v.at[j]], sem_s, add=True)
        pltpu.async_copy(ones_d, acc.at[dst_v.at[j]], sem_d, add=True)

        @pl.when(j >= WIN)
        def _():
            pltpu.make_async_copy(ones_s, acc.at[src_v.at[0]], sem_s).wait()
            pltpu.make_async_copy(ones_d, acc.at[dst_v.at[0]], sem_d).wait()

        return c

    lax.fori_loop(0, NCH, chunk, 0)
    for _ in range(WIN):
        pltpu.make_async_copy(ones_s, acc.at[src_v.at[0]], sem_s).wait()
        pltpu.make_async_copy(ones_d, acc.at[dst_v.at[0]], sem_d).wait()

    plsc.subcore_barrier()

    r0 = sid * RPS
    pltpu.sync_copy(acc.at[pl.ds(r0, RPS)], out_hbm.at[cid, pl.ds(r0, RPS)])


@functools.partial(
    pl.kernel,
    out_type=jax.ShapeDtypeStruct((NC, NP, D), jnp.float32),
    mesh=_MESH,
    compiler_params=_SC_PARAMS,
    scratch_types=[
        pltpu.VMEM((NCH, CH), jnp.int32),
        pltpu.VMEM((NCH, CH), jnp.int32),
        [[pltpu.VMEM((CH, D), jnp.float32)] * NB] * 2,
        pltpu.VMEM_SHARED((NP, D), jnp.float32),
        [[pltpu.SemaphoreType.DMA] * NB] * 2,
        [[pltpu.SemaphoreType.DMA] * NB] * 2,
    ],
)
def _agg(y_hbm, src_hbm, dst_hbm, out_hbm, src_v, dst_v, rows, acc,
         gsems, ssems):
    cid = lax.axis_index("c")
    sid = lax.axis_index("s")
    wid = cid * NS + sid

    _zero_vmem_f32(rows[0][0], CH, D)
    for k in range(RPS // CH):
        pltpu.sync_copy(rows[0][0], acc.at[pl.ds(sid * RPS + k * CH, CH)])

    pltpu.sync_copy(src_hbm.at[wid], src_v)
    pltpu.sync_copy(dst_hbm.at[wid], dst_v)

    plsc.subcore_barrier()

    def drain(p):
        for b in range(NB):
            pltpu.make_async_copy(rows[p][b], acc.at[dst_v.at[0]],
                                  ssems[p][b]).wait()

    def run_group(g, p):
        base = g * NB
        gds = [pltpu.async_copy(y_hbm.at[src_v.at[base + b]], rows[p][b],
                                gsems[p][b]) for b in range(NB)]
        for b in range(NB):
            gds[b].wait()
            pltpu.async_copy(rows[p][b], acc.at[dst_v.at[base + b]],
                             ssems[p][b], add=True)

    NG = NCH // NB

    def pair(i, c):
        for p in range(2):
            @pl.when(i > 0)
            def _():
                drain(p)

            run_group(2 * i + p, p)
        return c

    lax.fori_loop(0, NG // 2, pair, 0)
    drain(0)
    run_group(NG - 1, 0)
    drain(1)
    drain(0)

    plsc.subcore_barrier()

    r0 = sid * RPS
    pltpu.sync_copy(acc.at[pl.ds(r0, RPS)], out_hbm.at[cid, pl.ds(r0, RPS)])


_BLK = 2048
_GRID = NP // _BLK


def _tc0_body(hx_ref, degp_ref, w_ref, y_ref, nsd_ref):
    d = degp_ref[...]
    deg_o = d[0, :, 0] + d[1, :, 0]
    deg_i = d[0, :, 8] + d[1, :, 8]
    ns = lax.rsqrt(jnp.maximum(deg_o, 1.0))[:, None]
    nd = lax.rsqrt(jnp.maximum(deg_i, 1.0))[:, None]
    nsd_ref[...] = jnp.concatenate([ns, nd], axis=1)
    y_ref[...] = jnp.dot(hx_ref[...] * ns, w_ref[...],
                         preferred_element_type=jnp.float32)


def _tc0(hx, degp, W1):
    return pl.pallas_call(
        _tc0_body,
        out_shape=[jax.ShapeDtypeStruct((NP, D), jnp.float32),
                   jax.ShapeDtypeStruct((NP, 2), jnp.float32)],
        grid=(_GRID,),
        in_specs=[
            pl.BlockSpec((_BLK, 128), lambda i: (i, 0)),
            pl.BlockSpec((NC, _BLK, 16), lambda i: (0, i, 0)),
            pl.BlockSpec((128, D), lambda i: (0, 0)),
        ],
        out_specs=[pl.BlockSpec((_BLK, D), lambda i: (i, 0)),
                   pl.BlockSpec((_BLK, 2), lambda i: (i, 0))],
    )(hx, degp, W1)


def _layer_body(p_ref, w_ref, b_ref, nsd_ref, y_ref):
    p = p_ref[0] + p_ref[1]
    p = jnp.dot(p, w_ref[...], preferred_element_type=jnp.float32)
    nsd = nsd_ref[...]
    h = jnp.maximum(p * nsd[:, 1:2] + b_ref[...], 0.0)
    y_ref[...] = h * nsd[:, 0:1]


def _layer(partials, W, b, nsd):
    return pl.pallas_call(
        _layer_body,
        out_shape=jax.ShapeDtypeStruct((NP, D), jnp.float32),
        grid=(_GRID,),
        in_specs=[
            pl.BlockSpec((NC, _BLK, D), lambda i: (0, i, 0)),
            pl.BlockSpec((D, D), lambda i: (0, 0)),
            pl.BlockSpec((1, D), lambda i: (0, 0)),
            pl.BlockSpec((_BLK, 2), lambda i: (i, 0)),
        ],
        out_specs=pl.BlockSpec((_BLK, D), lambda i: (i, 0)),
    )(partials, W, b, nsd)


def _head_body(h_ref, nsd_ref, wp1_ref, bp1_ref, wp2_ref, bp2_ref, out_ref):
    rows = lax.broadcasted_iota(jnp.int32, (NP, 1), 0)
    h = jnp.where(rows < N, h_ref[...] / nsd_ref[:, 0:1], 0.0)
    m = jnp.sum(h, axis=0, keepdims=True) * (1.0 / N)
    t = jnp.maximum(jnp.dot(m, wp1_ref[...], preferred_element_type=jnp.float32)
                    + bp1_ref[...], 0.0)
    out_ref[...] = jnp.dot(t, wp2_ref[...], preferred_element_type=jnp.float32) \
        + bp2_ref[...]


def _head(h, nsd, Wp1, bp1, Wp2, bp2):
    return pl.pallas_call(
        _head_body,
        out_shape=jax.ShapeDtypeStruct((1, 10), jnp.float32),
    )(h, nsd, Wp1, bp1, Wp2, bp2)


def kernel(hx, edge_index, W1, b1, W2, b2, W3, b3, W4, b4, W5, b5, W6, b6,
           Wp1, bp1, Wp2, bp2):
    src3d = edge_index[0].reshape(NW, NCH, CH)
    dst3d = edge_index[1].reshape(NW, NCH, CH)
    hxp = jnp.concatenate(
        [hx, jnp.zeros((NP - N, hx.shape[1]), jnp.float32)], axis=0)

    degp = _hist(src3d, dst3d)
    y, nsd = _tc0(hxp, degp, W1)

    Wstack = jnp.stack([jnp.eye(D, dtype=jnp.float32), W2, W3, W4, W5, W6])
    bstack = jnp.stack([b.reshape(1, D) for b in (b1, b2, b3, b4, b5, b6)])

    def step(y, wb):
        W, b = wb
        partials = _agg(y, src3d, dst3d)
        return _layer(partials, W, b, nsd), None

    y, _ = lax.scan(step, y, (Wstack, bstack))

    return _head(y, nsd, Wp1, bp1.reshape(1, D // 2), Wp2, bp2.reshape(1, 10))


# speedup vs baseline: 2.8420x; 1.0654x over previous
import functools

import jax
import jax.numpy as jnp
from jax import lax
from jax.experimental import pallas as pl
from jax.experimental.pallas import tpu as pltpu
from jax.experimental.pallas import tpu_sc as plsc

N = 10000
NP = 10240
E = 320000
D = 64

NC = 2
NS = 16
NW = NC * NS
CH = 128
EP = 327680
NCH = EP // (NW * CH)
NB = 4
RPS = NP // NS

_MESH = plsc.VectorSubcoreMesh(core_axis_name="c", subcore_axis_name="s")
_SC_PARAMS = pltpu.CompilerParams(use_tc_tiling_on_sc=False)


def _zero_vmem_f32(ref, nrows, width):
    z16 = jnp.zeros((16,), jnp.float32)

    def body(i, c):
        for j in range(width // 16):
            ref[i, pl.ds(j * 16, 16)] = z16
        return c

    lax.fori_loop(0, nrows, body, 0)


@functools.partial(
    pl.kernel,
    out_type=jax.ShapeDtypeStruct((NC, NP, 16), jnp.float32),
    mesh=_MESH,
    compiler_params=_SC_PARAMS,
    scratch_types=[
        pltpu.VMEM((NCH, CH), jnp.int32),
        pltpu.VMEM((NCH, CH), jnp.int32),
        pltpu.VMEM((CH, 16), jnp.float32),
        pltpu.VMEM((CH, 16), jnp.float32),
        pltpu.VMEM((RPS // 5, 16), jnp.float32),
        pltpu.VMEM_SHARED((NP, 16), jnp.float32),
        pltpu.SemaphoreType.DMA,
        pltpu.SemaphoreType.DMA,
    ],
)
def _hist(src_hbm, dst_hbm, out_hbm, src_v, dst_v, ones_s, ones_d, zbuf, acc,
          sem_s, sem_d):
    cid = lax.axis_index("c")
    sid = lax.axis_index("s")
    wid = cid * NS + sid

    lane = lax.iota(jnp.int32, 16)
    row_s = jnp.where(lane < 8, 1.0, 0.0)
    row_d = jnp.where(lane < 8, 0.0, 1.0)

    def fill_ones(i, c):
        ones_s[i, pl.ds(0, 16)] = row_s
        ones_d[i, pl.ds(0, 16)] = row_d
        return c

    lax.fori_loop(0, CH, fill_ones, 0)
    _zero_vmem_f32(zbuf, RPS // 5, 16)

    for k in range(5):
        pltpu.sync_copy(
            zbuf, acc.at[pl.ds(sid * RPS + k * (RPS // 5), RPS // 5)])

    pltpu.sync_copy(src_hbm.at[wid], src_v)
    pltpu.sync_copy(dst_hbm.at[wid], dst_v)

    plsc.subcore_barrier()

    WIN = 4

    def chunk(j, c):
        pltpu.async_copy(ones_s, acc.at[src_v.at[j]], sem_s, add=True)
        pltpu.async_copy(ones_d, acc.at[dst_v.at[j]], sem_d, add=True)

        @pl.when(j >= WIN)
        def _():
            pltpu.make_async_copy(ones_s, acc.at[src_v.at[0]], sem_s).wait()
            pltpu.make_async_copy(ones_d, acc.at[dst_v.at[0]], sem_d).wait()

        return c

    lax.fori_loop(0, NCH, chunk, 0)
    for _ in range(WIN):
        pltpu.make_async_copy(ones_s, acc.at[src_v.at[0]], sem_s).wait()
        pltpu.make_async_copy(ones_d, acc.at[dst_v.at[0]], sem_d).wait()

    plsc.subcore_barrier()

    r0 = sid * RPS
    pltpu.sync_copy(acc.at[pl.ds(r0, RPS)], out_hbm.at[cid, pl.ds(r0, RPS)])


@functools.partial(
    pl.kernel,
    out_type=jax.ShapeDtypeStruct((NC, NP, D), jnp.float32),
    mesh=_MESH,
    compiler_params=_SC_PARAMS,
    scratch_types=[
        pltpu.VMEM((NCH, CH), jnp.int32),
        pltpu.VMEM((NCH, CH), jnp.int32),
        [[pltpu.VMEM((CH, D), jnp.float32)] * NB] * 2,
        pltpu.VMEM_SHARED((NP, D), jnp.float32),
        [[pltpu.SemaphoreType.DMA] * NB] * 2,
        [[pltpu.SemaphoreType.DMA] * NB] * 2,
    ],
)
def _agg(y_hbm, src_hbm, dst_hbm, out_hbm, src_v, dst_v, rows, acc,
         gsems, ssems):
    cid = lax.axis_index("c")
    sid = lax.axis_index("s")
    wid = cid * NS + sid

    _zero_vmem_f32(rows[0][0], CH, D)
    for k in range(RPS // CH):
        pltpu.sync_copy(rows[0][0], acc.at[pl.ds(sid * RPS + k * CH, CH)])


    pltpu.sync_copy(src_hbm.at[wid], src_v)
    pltpu.sync_copy(dst_hbm.at[wid], dst_v)

    plsc.subcore_barrier()

    def drain(p):
        for b in range(NB):
            pltpu.make_async_copy(rows[p][b], acc.at[dst_v.at[0]],
                                  ssems[p][b]).wait()

    def run_group(g, p):
        base = g * NB
        gds = [pltpu.async_copy(y_hbm.at[src_v.at[base + b]], rows[p][b],
                                gsems[p][b]) for b in range(NB)]
        for b in range(NB):
            gds[b].wait()
            pltpu.async_copy(rows[p][b], acc.at[dst_v.at[base + b]],
                             ssems[p][b], add=True)

    NG = NCH // NB

    def pair(i, c):
        for p in range(2):
            @pl.when(i > 0)
            def _():
                drain(p)

            run_group(2 * i + p, p)
        return c

    lax.fori_loop(0, NG // 2, pair, 0)
    drain(0)
    drain(1)

    plsc.subcore_barrier()

    r0 = sid * RPS
    pltpu.sync_copy(acc.at[pl.ds(r0, RPS)], out_hbm.at[cid, pl.ds(r0, RPS)])


_BLK = 2048
_GRID = NP // _BLK


def _tc0_body(hx_ref, degp_ref, w_ref, y_ref, nsd_ref):
    d = degp_ref[...]
    deg_o = d[0, :, 0] + d[1, :, 0]
    deg_i = d[0, :, 8] + d[1, :, 8]
    ns = lax.rsqrt(jnp.maximum(deg_o, 1.0))[:, None]
    nd = lax.rsqrt(jnp.maximum(deg_i, 1.0))[:, None]
    nsd_ref[...] = jnp.concatenate([ns, nd], axis=1)
    y_ref[...] = jnp.dot(hx_ref[...] * ns, w_ref[...],
                         preferred_element_type=jnp.float32)


def _tc0(hx, degp, W1):
    return pl.pallas_call(
        _tc0_body,
        out_shape=[jax.ShapeDtypeStruct((NP, D), jnp.float32),
                   jax.ShapeDtypeStruct((NP, 2), jnp.float32)],
        grid=(_GRID,),
        in_specs=[
            pl.BlockSpec((_BLK, 128), lambda i: (i, 0)),
            pl.BlockSpec((NC, _BLK, 16), lambda i: (0, i, 0)),
            pl.BlockSpec((128, D), lambda i: (0, 0)),
        ],
        out_specs=[pl.BlockSpec((_BLK, D), lambda i: (i, 0)),
                   pl.BlockSpec((_BLK, 2), lambda i: (i, 0))],
    )(hx, degp, W1)


def _layer_body(p_ref, w_ref, b_ref, nsd_ref, y_ref):
    p = p_ref[0] + p_ref[1]
    p = jnp.dot(p, w_ref[...], preferred_element_type=jnp.float32)
    nsd = nsd_ref[...]
    h = jnp.maximum(p * nsd[:, 1:2] + b_ref[...], 0.0)
    y_ref[...] = h * nsd[:, 0:1]


def _layer(partials, W, b, nsd):
    return pl.pallas_call(
        _layer_body,
        out_shape=jax.ShapeDtypeStruct((NP, D), jnp.float32),
        grid=(_GRID,),
        in_specs=[
            pl.BlockSpec((NC, _BLK, D), lambda i: (0, i, 0)),
            pl.BlockSpec((D, D), lambda i: (0, 0)),
            pl.BlockSpec((1, D), lambda i: (0, 0)),
            pl.BlockSpec((_BLK, 2), lambda i: (i, 0)),
        ],
        out_specs=pl.BlockSpec((_BLK, D), lambda i: (i, 0)),
    )(partials, W, b, nsd)


def _head_body(h_ref, nsd_ref, wp1_ref, bp1_ref, wp2_ref, bp2_ref, out_ref):
    rows = lax.broadcasted_iota(jnp.int32, (NP, 1), 0)
    h = jnp.where(rows < N, h_ref[...] / nsd_ref[:, 0:1], 0.0)
    m = jnp.sum(h, axis=0, keepdims=True) * (1.0 / N)
    t = jnp.maximum(jnp.dot(m, wp1_ref[...], preferred_element_type=jnp.float32)
                    + bp1_ref[...], 0.0)
    out_ref[...] = jnp.dot(t, wp2_ref[...], preferred_element_type=jnp.float32) \
        + bp2_ref[...]


def _head(h, nsd, Wp1, bp1, Wp2, bp2):
    return pl.pallas_call(
        _head_body,
        out_shape=jax.ShapeDtypeStruct((1, 10), jnp.float32),
    )(h, nsd, Wp1, bp1, Wp2, bp2)


def kernel(hx, edge_index, W1, b1, W2, b2, W3, b3, W4, b4, W5, b5, W6, b6,
           Wp1, bp1, Wp2, bp2):
    spread = N + (jnp.arange(EP - E, dtype=jnp.int32) % (NP - N))
    src3d = jnp.concatenate([edge_index[0], spread]).reshape(NW, NCH, CH)
    dst3d = jnp.concatenate([edge_index[1], spread]).reshape(NW, NCH, CH)
    hxp = jnp.concatenate(
        [hx, jnp.zeros((NP - N, hx.shape[1]), jnp.float32)], axis=0)

    degp = _hist(src3d, dst3d)
    y, nsd = _tc0(hxp, degp, W1)

    Wstack = jnp.stack([jnp.eye(D, dtype=jnp.float32), W2, W3, W4, W5, W6])
    bstack = jnp.stack([b.reshape(1, D) for b in (b1, b2, b3, b4, b5, b6)])

    def step(y, wb):
        W, b = wb
        partials = _agg(y, src3d, dst3d)
        return _layer(partials, W, b, nsd), None

    y, _ = lax.scan(step, y, (Wstack, bstack))

    return _head(y, nsd, Wp1, bp1.reshape(1, D // 2), Wp2, bp2.reshape(1, 10))


# speedup vs baseline: 2.9101x; 1.0240x over previous
import functools

import jax
import jax.numpy as jnp
from jax import lax
from jax.experimental import pallas as pl
from jax.experimental.pallas import tpu as pltpu
from jax.experimental.pallas import tpu_sc as plsc

N = 10000
NP = 10240
E = 320000
D = 64

NC = 2
NS = 16
NW = NC * NS
CH = 128
EP = 327680
NCH = EP // (NW * CH)
NB = 4
RPS = NP // NS

_MESH = plsc.VectorSubcoreMesh(core_axis_name="c", subcore_axis_name="s")
_SC_PARAMS = pltpu.CompilerParams(use_tc_tiling_on_sc=False)


def _zero_vmem_f32(ref, nrows, width):
    z16 = jnp.zeros((16,), jnp.float32)

    def body(i, c):
        for j in range(width // 16):
            ref[i, pl.ds(j * 16, 16)] = z16
        return c

    lax.fori_loop(0, nrows, body, 0)


@functools.partial(
    pl.kernel,
    out_type=jax.ShapeDtypeStruct((NC, NP, 16), jnp.float32),
    mesh=_MESH,
    compiler_params=_SC_PARAMS,
    scratch_types=[
        pltpu.VMEM((NCH, CH), jnp.int32),
        pltpu.VMEM((NCH, CH), jnp.int32),
        pltpu.VMEM((CH, 16), jnp.float32),
        pltpu.VMEM((CH, 16), jnp.float32),
        pltpu.VMEM((RPS // 5, 16), jnp.float32),
        pltpu.VMEM_SHARED((NP, 16), jnp.float32),
        pltpu.SemaphoreType.DMA,
        pltpu.SemaphoreType.DMA,
    ],
)
def _hist(src_hbm, dst_hbm, out_hbm, src_v, dst_v, ones_s, ones_d, zbuf, acc,
          sem_s, sem_d):
    cid = lax.axis_index("c")
    sid = lax.axis_index("s")
    wid = cid * NS + sid

    lane = lax.iota(jnp.int32, 16)
    row_s = jnp.where(lane < 8, 1.0, 0.0)
    row_d = jnp.where(lane < 8, 0.0, 1.0)

    def fill_ones(i, c):
        ones_s[i, pl.ds(0, 16)] = row_s
        ones_d[i, pl.ds(0, 16)] = row_d
        return c

    lax.fori_loop(0, CH, fill_ones, 0)
    _zero_vmem_f32(zbuf, RPS // 5, 16)

    for k in range(5):
        pltpu.sync_copy(
            zbuf, acc.at[pl.ds(sid * RPS + k * (RPS // 5), RPS // 5)])

    pltpu.sync_copy(src_hbm.at[wid], src_v)
    pltpu.sync_copy(dst_hbm.at[wid], dst_v)

    plsc.subcore_barrier()

    WIN = 4

    def chunk(j, c):
        pltpu.async_copy(ones_s, acc.at[src_v.at[j]], sem_s, add=True)
        pltpu.async_copy(ones_d, acc.at[dst_v.at[j]], sem_d, add=True)

        @pl.when(j >= WIN)
        def _():
            pltpu.make_async_copy(ones_s, acc.at[src_v.at[0]], sem_s).wait()
            pltpu.make_async_copy(ones_d, acc.at[dst_v.at[0]], sem_d).wait()

        return c

    lax.fori_loop(0, NCH, chunk, 0)
    for _ in range(WIN):
        pltpu.make_async_copy(ones_s, acc.at[src_v.at[0]], sem_s).wait()
        pltpu.make_async_copy(ones_d, acc.at[dst_v.at[0]], sem_d).wait()

    plsc.subcore_barrier()

    r0 = sid * RPS
    pltpu.sync_copy(acc.at[pl.ds(r0, RPS)], out_hbm.at[cid, pl.ds(r0, RPS)])


@functools.partial(
    pl.kernel,
    out_type=jax.ShapeDtypeStruct((NC, NP, D), jnp.float32),
    mesh=_MESH,
    compiler_params=_SC_PARAMS,
    scratch_types=[
        pltpu.VMEM((NCH, CH), jnp.int32),
        pltpu.VMEM((NCH, CH), jnp.int32),
        [[pltpu.VMEM((CH, D), jnp.float32)] * NB] * 2,
        pltpu.VMEM_SHARED((NP, D), jnp.float32),
        [[pltpu.SemaphoreType.DMA] * NB] * 2,
        [[pltpu.SemaphoreType.DMA] * NB] * 2,
    ],
)
def _agg(y_hbm, src_hbm, dst_hbm, out_hbm, src_v, dst_v, rows, acc,
         gsems, ssems):
    cid = lax.axis_index("c")
    sid = lax.axis_index("s")
    wid = cid * NS + sid

    ld_s = pltpu.async_copy(src_hbm.at[wid], src_v, gsems[0][0])
    ld_d = pltpu.async_copy(dst_hbm.at[wid], dst_v, gsems[0][1])
    _zero_vmem_f32(rows[0][0], CH, D)
    for k in range(RPS // CH):
        pltpu.async_copy(rows[0][0], acc.at[pl.ds(sid * RPS + k * CH, CH)],
                         ssems[0][0])
    ld_s.wait()
    ld_d.wait()
    for k in range(RPS // CH):
        pltpu.make_async_copy(rows[0][0], acc.at[pl.ds(sid * RPS, CH)],
                              ssems[0][0]).wait()

    plsc.subcore_barrier()

    def drain(p):
        for b in range(NB):
            pltpu.make_async_copy(rows[p][b], acc.at[dst_v.at[0]],
                                  ssems[p][b]).wait()

    def run_group(g, p):
        base = g * NB
        gds = [pltpu.async_copy(y_hbm.at[src_v.at[base + b]], rows[p][b],
                                gsems[p][b]) for b in range(NB)]
        for b in range(NB):
            gds[b].wait()
            pltpu.async_copy(rows[p][b], acc.at[dst_v.at[base + b]],
                             ssems[p][b], add=True)

    NG = NCH // NB

    def pair(i, c):
        for p in range(2):
            @pl.when(i > 0)
            def _():
                drain(p)

            run_group(2 * i + p, p)
        return c

    lax.fori_loop(0, NG // 2, pair, 0)
    drain(0)
    drain(1)

    plsc.subcore_barrier()

    r0 = sid * RPS
    pltpu.sync_copy(acc.at[pl.ds(r0, RPS)], out_hbm.at[cid, pl.ds(r0, RPS)])


_BLK = 2048
_GRID = NP // _BLK


def _mm_body(hx_ref, w_ref, z_ref):
    z_ref[...] = jnp.dot(hx_ref[...], w_ref[...],
                         preferred_element_type=jnp.float32)


def _mm(hx, W1):
    return pl.pallas_call(
        _mm_body,
        out_shape=jax.ShapeDtypeStruct((NP, D), jnp.float32),
        grid=(_GRID,),
        in_specs=[
            pl.BlockSpec((_BLK, 128), lambda i: (i, 0)),
            pl.BlockSpec((128, D), lambda i: (0, 0)),
        ],
        out_specs=pl.BlockSpec((_BLK, D), lambda i: (i, 0)),
    )(hx, W1)


def _tc0_body(z_ref, degp_ref, y_ref, nsd_ref):
    d = degp_ref[...]
    deg_o = d[0, :, 0] + d[1, :, 0]
    deg_i = d[0, :, 8] + d[1, :, 8]
    ns = lax.rsqrt(jnp.maximum(deg_o, 1.0))[:, None]
    nd = lax.rsqrt(jnp.maximum(deg_i, 1.0))[:, None]
    nsd_ref[...] = jnp.concatenate([ns, nd], axis=1)
    y_ref[...] = z_ref[...] * ns


def _tc0(z, degp):
    return pl.pallas_call(
        _tc0_body,
        out_shape=[jax.ShapeDtypeStruct((NP, D), jnp.float32),
                   jax.ShapeDtypeStruct((NP, 2), jnp.float32)],
        grid=(_GRID,),
        in_specs=[
            pl.BlockSpec((_BLK, D), lambda i: (i, 0)),
            pl.BlockSpec((NC, _BLK, 16), lambda i: (0, i, 0)),
        ],
        out_specs=[pl.BlockSpec((_BLK, D), lambda i: (i, 0)),
                   pl.BlockSpec((_BLK, 2), lambda i: (i, 0))],
    )(z, degp)


def _layer_body(p_ref, w_ref, b_ref, nsd_ref, y_ref):
    p = p_ref[0] + p_ref[1]
    p = jnp.dot(p, w_ref[...], preferred_element_type=jnp.float32)
    nsd = nsd_ref[...]
    h = jnp.maximum(p * nsd[:, 1:2] + b_ref[...], 0.0)
    y_ref[...] = h * nsd[:, 0:1]


def _layer(partials, W, b, nsd):
    return pl.pallas_call(
        _layer_body,
        out_shape=jax.ShapeDtypeStruct((NP, D), jnp.float32),
        grid=(_GRID,),
        in_specs=[
            pl.BlockSpec((NC, _BLK, D), lambda i: (0, i, 0)),
            pl.BlockSpec((D, D), lambda i: (0, 0)),
            pl.BlockSpec((1, D), lambda i: (0, 0)),
            pl.BlockSpec((_BLK, 2), lambda i: (i, 0)),
        ],
        out_specs=pl.BlockSpec((_BLK, D), lambda i: (i, 0)),
    )(partials, W, b, nsd)


def _head_body(h_ref, nsd_ref, wp1_ref, bp1_ref, wp2_ref, bp2_ref, out_ref):
    rows = lax.broadcasted_iota(jnp.int32, (NP, 1), 0)
    h = jnp.where(rows < N, h_ref[...] / nsd_ref[:, 0:1], 0.0)
    m = jnp.sum(h, axis=0, keepdims=True) * (1.0 / N)
    t = jnp.maximum(jnp.dot(m, wp1_ref[...], preferred_element_type=jnp.float32)
                    + bp1_ref[...], 0.0)
    out_ref[...] = jnp.dot(t, wp2_ref[...], preferred_element_type=jnp.float32) \
        + bp2_ref[...]


def _head(h, nsd, Wp1, bp1, Wp2, bp2):
    return pl.pallas_call(
        _head_body,
        out_shape=jax.ShapeDtypeStruct((1, 10), jnp.float32),
    )(h, nsd, Wp1, bp1, Wp2, bp2)


def kernel(hx, edge_index, W1, b1, W2, b2, W3, b3, W4, b4, W5, b5, W6, b6,
           Wp1, bp1, Wp2, bp2):
    spread = N + (jnp.arange(EP - E, dtype=jnp.int32) % (NP - N))
    src3d = jnp.concatenate([edge_index[0], spread]).reshape(NW, NCH, CH)
    dst3d = jnp.concatenate([edge_index[1], spread]).reshape(NW, NCH, CH)
    hxp = jnp.concatenate(
        [hx, jnp.zeros((NP - N, hx.shape[1]), jnp.float32)], axis=0)

    z = _mm(hxp, W1)
    degp = _hist(src3d, dst3d)
    y, nsd = _tc0(z, degp)

    Wstack = jnp.stack([jnp.eye(D, dtype=jnp.float32), W2, W3, W4, W5, W6])
    bstack = jnp.stack([b.reshape(1, D) for b in (b1, b2, b3, b4, b5, b6)])

    def step(y, wb):
        W, b = wb
        partials = _agg(y, src3d, dst3d)
        return _layer(partials, W, b, nsd), None

    y, _ = lax.scan(step, y, (Wstack, bstack))

    return _head(y, nsd, Wp1, bp1.reshape(1, D // 2), Wp2, bp2.reshape(1, 10))


# speedup vs baseline: 2.9123x; 1.0007x over previous
import functools

import jax
import jax.numpy as jnp
from jax import lax
from jax.experimental import pallas as pl
from jax.experimental.pallas import tpu as pltpu
from jax.experimental.pallas import tpu_sc as plsc

N = 10000
NP = 10240
E = 320000
D = 64

NC = 2
NS = 16
NW = NC * NS
CH = 128
EP = 327680
NCH = EP // (NW * CH)
NB = 4
RPS = NP // NS

_MESH = plsc.VectorSubcoreMesh(core_axis_name="c", subcore_axis_name="s")
_SC_PARAMS = pltpu.CompilerParams(use_tc_tiling_on_sc=False)


def _zero_vmem_f32(ref, nrows, width):
    z16 = jnp.zeros((16,), jnp.float32)

    def body(i, c):
        for j in range(width // 16):
            ref[i, pl.ds(j * 16, 16)] = z16
        return c

    lax.fori_loop(0, nrows, body, 0)


@functools.partial(
    pl.kernel,
    out_type=jax.ShapeDtypeStruct((NC, NP, 8), jnp.float32),
    mesh=_MESH,
    compiler_params=_SC_PARAMS,
    scratch_types=[
        pltpu.VMEM((NCH, CH), jnp.int32),
        pltpu.VMEM((NCH, CH), jnp.int32),
        pltpu.VMEM((CH, 8), jnp.float32),
        pltpu.VMEM((CH, 8), jnp.float32),
        pltpu.VMEM((RPS // 5, 8), jnp.float32),
        pltpu.VMEM_SHARED((NP, 8), jnp.float32),
        pltpu.SemaphoreType.DMA,
        pltpu.SemaphoreType.DMA,
    ],
)
def _hist(src_hbm, dst_hbm, const_hbm, out_hbm, src_v, dst_v, ones_s, ones_d,
          zbuf, acc, sem_s, sem_d):
    cid = lax.axis_index("c")
    sid = lax.axis_index("s")
    wid = cid * NS + sid

    pltpu.sync_copy(const_hbm.at[pl.ds(0, CH)], ones_s)
    pltpu.sync_copy(const_hbm.at[pl.ds(CH, CH)], ones_d)
    pltpu.sync_copy(const_hbm.at[pl.ds(2 * CH, RPS // 5)], zbuf)

    for k in range(5):
        pltpu.sync_copy(
            zbuf, acc.at[pl.ds(sid * RPS + k * (RPS // 5), RPS // 5)])

    pltpu.sync_copy(src_hbm.at[wid], src_v)
    pltpu.sync_copy(dst_hbm.at[wid], dst_v)

    plsc.subcore_barrier()

    WIN = 8

    def chunk(j, c):
        pltpu.async_copy(ones_s, acc.at[src_v.at[j]], sem_s, add=True)
        pltpu.async_copy(ones_d, acc.at[dst_v.at[j]], sem_d, add=True)

        @pl.when(j >= WIN)
        def _():
            pltpu.make_async_copy(ones_s, acc.at[src_v.at[0]], sem_s).wait()
            pltpu.make_async_copy(ones_d, acc.at[dst_v.at[0]], sem_d).wait()

        return c

    lax.fori_loop(0, NCH, chunk, 0)
    for _ in range(WIN):
        pltpu.make_async_copy(ones_s, acc.at[src_v.at[0]], sem_s).wait()
        pltpu.make_async_copy(ones_d, acc.at[dst_v.at[0]], sem_d).wait()

    plsc.subcore_barrier()

    r0 = sid * RPS
    pltpu.sync_copy(acc.at[pl.ds(r0, RPS)], out_hbm.at[cid, pl.ds(r0, RPS)])


@functools.partial(
    pl.kernel,
    out_type=jax.ShapeDtypeStruct((NC, NP, D), jnp.float32),
    mesh=_MESH,
    compiler_params=_SC_PARAMS,
    scratch_types=[
        pltpu.VMEM((NCH, CH), jnp.int32),
        pltpu.VMEM((NCH, CH), jnp.int32),
        [[pltpu.VMEM((CH, D), jnp.float32)] * NB] * 2,
        pltpu.VMEM_SHARED((NP, D), jnp.float32),
        [[pltpu.SemaphoreType.DMA] * NB] * 2,
        [[pltpu.SemaphoreType.DMA] * NB] * 2,
    ],
)
def _agg(y_hbm, src_hbm, dst_hbm, out_hbm, src_v, dst_v, rows, acc,
         gsems, ssems):
    cid = lax.axis_index("c")
    sid = lax.axis_index("s")
    wid = cid * NS + sid

    ld_s = pltpu.async_copy(src_hbm.at[wid], src_v, gsems[0][0])
    ld_d = pltpu.async_copy(dst_hbm.at[wid], dst_v, gsems[0][1])
    _zero_vmem_f32(rows[0][0], CH, D)
    for k in range(RPS // CH):
        pltpu.async_copy(rows[0][0], acc.at[pl.ds(sid * RPS + k * CH, CH)],
                         ssems[0][0])
    ld_s.wait()
    ld_d.wait()
    for k in range(RPS // CH):
        pltpu.make_async_copy(rows[0][0], acc.at[pl.ds(sid * RPS, CH)],
                              ssems[0][0]).wait()

    plsc.subcore_barrier()

    def drain(p):
        for b in range(NB):
            pltpu.make_async_copy(rows[p][b], acc.at[dst_v.at[0]],
                                  ssems[p][b]).wait()

    def run_group(g, p):
        base = g * NB
        gds = [pltpu.async_copy(y_hbm.at[src_v.at[base + b]], rows[p][b],
                                gsems[p][b]) for b in range(NB)]
        for b in range(NB):
            gds[b].wait()
            pltpu.async_copy(rows[p][b], acc.at[dst_v.at[base + b]],
                             ssems[p][b], add=True)

    NG = NCH // NB

    def pair(i, c):
        for p in range(2):
            @pl.when(i > 0)
            def _():
                drain(p)

            run_group(2 * i + p, p)
        return c

    lax.fori_loop(0, NG // 2, pair, 0)
    drain(0)
    drain(1)

    plsc.subcore_barrier()

    r0 = sid * RPS
    pltpu.sync_copy(acc.at[pl.ds(r0, RPS)], out_hbm.at[cid, pl.ds(r0, RPS)])


_BLK = 2048
_GRID = NP // _BLK


def _mm_body(hx_ref, w_ref, z_ref):
    z_ref[...] = jnp.dot(hx_ref[...], w_ref[...],
                         preferred_element_type=jnp.float32)


def _mm(hx, W1):
    return pl.pallas_call(
        _mm_body,
        out_shape=jax.ShapeDtypeStruct((NP, D), jnp.float32),
        grid=(_GRID,),
        in_specs=[
            pl.BlockSpec((_BLK, 128), lambda i: (i, 0)),
            pl.BlockSpec((128, D), lambda i: (0, 0)),
        ],
        out_specs=pl.BlockSpec((_BLK, D), lambda i: (i, 0)),
    )(hx, W1)


def _tc0_body(z_ref, degp_ref, y_ref, nsd_ref):
    d = degp_ref[...]
    deg_o = d[0, :, 0] + d[1, :, 0]
    deg_i = d[0, :, 4] + d[1, :, 4]
    ns = lax.rsqrt(jnp.maximum(deg_o, 1.0))[:, None]
    nd = lax.rsqrt(jnp.maximum(deg_i, 1.0))[:, None]
    nsd_ref[...] = jnp.concatenate([ns, nd], axis=1)
    y_ref[...] = z_ref[...] * ns


def _tc0(z, degp):
    return pl.pallas_call(
        _tc0_body,
        out_shape=[jax.ShapeDtypeStruct((NP, D), jnp.float32),
                   jax.ShapeDtypeStruct((NP, 2), jnp.float32)],
        grid=(_GRID,),
        in_specs=[
            pl.BlockSpec((_BLK, D), lambda i: (i, 0)),
            pl.BlockSpec((NC, _BLK, 8), lambda i: (0, i, 0)),
        ],
        out_specs=[pl.BlockSpec((_BLK, D), lambda i: (i, 0)),
                   pl.BlockSpec((_BLK, 2), lambda i: (i, 0))],
    )(z, degp)


def _layer_body(p_ref, w_ref, b_ref, nsd_ref, y_ref):
    p = p_ref[0] + p_ref[1]
    p = jnp.dot(p, w_ref[...], preferred_element_type=jnp.float32)
    nsd = nsd_ref[...]
    h = jnp.maximum(p * nsd[:, 1:2] + b_ref[...], 0.0)
    y_ref[...] = h * nsd[:, 0:1]


def _layer(partials, W, b, nsd):
    return pl.pallas_call(
        _layer_body,
        out_shape=jax.ShapeDtypeStruct((NP, D), jnp.float32),
        grid=(_GRID,),
        in_specs=[
            pl.BlockSpec((NC, _BLK, D), lambda i: (0, i, 0)),
            pl.BlockSpec((D, D), lambda i: (0, 0)),
            pl.BlockSpec((1, D), lambda i: (0, 0)),
            pl.BlockSpec((_BLK, 2), lambda i: (i, 0)),
        ],
        out_specs=pl.BlockSpec((_BLK, D), lambda i: (i, 0)),
    )(partials, W, b, nsd)


def _head_body(h_ref, nsd_ref, wp1_ref, bp1_ref, wp2_ref, bp2_ref, out_ref):
    rows = lax.broadcasted_iota(jnp.int32, (NP, 1), 0)
    h = jnp.where(rows < N, h_ref[...] / nsd_ref[:, 0:1], 0.0)
    m = jnp.sum(h, axis=0, keepdims=True) * (1.0 / N)
    t = jnp.maximum(jnp.dot(m, wp1_ref[...], preferred_element_type=jnp.float32)
                    + bp1_ref[...], 0.0)
    out_ref[...] = jnp.dot(t, wp2_ref[...], preferred_element_type=jnp.float32) \
        + bp2_ref[...]


def _head(h, nsd, Wp1, bp1, Wp2, bp2):
    return pl.pallas_call(
        _head_body,
        out_shape=jax.ShapeDtypeStruct((1, 10), jnp.float32),
    )(h, nsd, Wp1, bp1, Wp2, bp2)


def kernel(hx, edge_index, W1, b1, W2, b2, W3, b3, W4, b4, W5, b5, W6, b6,
           Wp1, bp1, Wp2, bp2):
    spread = N + (jnp.arange(EP - E, dtype=jnp.int32) % (NP - N))
    src3d = jnp.concatenate([edge_index[0], spread]).reshape(NW, NCH, CH)
    dst3d = jnp.concatenate([edge_index[1], spread]).reshape(NW, NCH, CH)
    hxp = jnp.concatenate(
        [hx, jnp.zeros((NP - N, hx.shape[1]), jnp.float32)], axis=0)

    z = _mm(hxp, W1)
    col = jnp.arange(8)
    consts = jnp.concatenate([
        jnp.broadcast_to((col < 4).astype(jnp.float32), (CH, 8)),
        jnp.broadcast_to(((col >= 4) & (col < 8)).astype(jnp.float32), (CH, 8)),
        jnp.zeros((RPS // 5, 8), jnp.float32),
    ])
    degp = _hist(src3d, dst3d, consts)
    y, nsd = _tc0(z, degp)

    Wstack = jnp.stack([jnp.eye(D, dtype=jnp.float32), W2, W3, W4, W5, W6])
    bstack = jnp.stack([b.reshape(1, D) for b in (b1, b2, b3, b4, b5, b6)])

    def step(y, wb):
        W, b = wb
        partials = _agg(y, src3d, dst3d)
        return _layer(partials, W, b, nsd), None

    y, _ = lax.scan(step, y, (Wstack, bstack))

    return _head(y, nsd, Wp1, bp1.reshape(1, D // 2), Wp2, bp2.reshape(1, 10))
